# Initial kernel scaffold; baseline (speedup 1.0000x reference)
import jax
import jax.numpy as jnp
from jax.experimental import pallas as pl


def kernel(x, W_lin, bias, E):
    raise NotImplementedError("write your pallas kernel here")


# speedup vs baseline: 52.8417x; 52.8417x over previous
import functools
import numpy as np
import jax
import jax.numpy as jnp
from jax import lax
from jax.experimental import pallas as pl
from jax.experimental.pallas import tpu as pltpu
from jax.experimental.pallas import tpu_sc as plsc

F = 26
D = 16
V = 26000
B = 4096
NC, NS = 2, 16
NW = NC * NS
BPW = B // NW
C = 4
NCH = BPW // C
NPAIR = F * (F - 1) // 2
EPS = 2 * NPAIR
EPC = C * EPS
NSTR = 21
EP = NSTR * 128


def _build_tables():
    pairs = [(i, j) for i in range(F) for j in range(i + 1, F)]
    tb = np.zeros(EP, np.int32)
    tf = np.zeros(EP, np.int32)
    for b in range(C):
        for p, (i, j) in enumerate(pairs):
            e = b * EPS + 2 * p
            tb[e] = b * F + i
            tf[e] = j * V + 1000 * i
            tb[e + 1] = b * F + j
            tf[e + 1] = i * V + 1000 * j
    lbi = np.zeros(C * 32, np.int32)
    loff = np.zeros(C * 32, np.int32)
    for b in range(C):
        for i in range(F):
            lbi[b * 32 + i] = b * F + i
            loff[b * 32 + i] = 1000 * i
    return tb, tf, lbi, loff


_TB, _TF, _LBI, _LOFF = _build_tables()

@functools.cache
def _build_sc_kernel():
    mesh = plsc.VectorSubcoreMesh(core_axis_name="c", subcore_axis_name="s",
                                  num_cores=NC, num_subcores=NS)
    return functools.partial(
        pl.kernel,
        out_type=jax.ShapeDtypeStruct((B,), jnp.float32),
        mesh=mesh,
        compiler_params=pltpu.CompilerParams(needs_layout_passes=False,
                                             use_tc_tiling_on_sc=False),
        scratch_types=[
        pltpu.VMEM((BPW * F,), jnp.int32),
        pltpu.VMEM((EP,), jnp.int32),
        pltpu.VMEM((EP,), jnp.int32),
        pltpu.VMEM((C * 32,), jnp.int32),
        pltpu.VMEM((C * 32,), jnp.int32),
        pltpu.VMEM((NSTR, 128), jnp.int32),
        pltpu.VMEM((EP, D), jnp.float32),
        pltpu.VMEM((C * 32,), jnp.int32),
        pltpu.VMEM((C * 32,), jnp.float32),
        pltpu.VMEM((BPW,), jnp.float32),
        pltpu.VMEM((16,), jnp.float32),
            pltpu.SemaphoreType.DMA,
            pltpu.SemaphoreType.DMA,
        ],
    )(_ffm_body)


def _ffm_body(x_hbm, w_hbm, b_hbm, e_hbm, tb_hbm, tf_hbm, lbi_hbm, loff_hbm,
            out_hbm, x_v, tb_v, tf_v, lbi_v, loff_v, idx_v, rows_v, li_v,
            wv_v, o_v, bias_v, sem, lsem):
    wid = lax.axis_index("s") * NC + lax.axis_index("c")
    base = wid * BPW

    pltpu.sync_copy(x_hbm.at[pl.ds(base * F, BPW * F)], x_v)
    pltpu.sync_copy(tb_hbm, tb_v)
    pltpu.sync_copy(tf_hbm, tf_v)
    pltpu.sync_copy(lbi_hbm, lbi_v)
    pltpu.sync_copy(loff_hbm, loff_v)
    pltpu.sync_copy(b_hbm, bias_v.at[pl.ds(0, 1)])

    lane = lax.iota(jnp.int32, 16)
    lmask = jnp.where(lane < (F - 16), 1.0, 0.0).astype(jnp.float32)
    zero = jnp.zeros((D,), jnp.float32)
    bvec = jnp.where(lane < 1, bias_v[pl.ds(0, 16)], 0.0)

    def chunk_body(c, gv):
        xoff = c * (C * F)

        def gen_body(r, _):
            for kk in range(8):
                s = r * 128 + kk * 16
                bi = tb_v[pl.ds(s, 16)] + xoff
                xv = plsc.load_gather(x_v, [bi])
                idx_v[r, pl.ds(kk * 16, 16)] = xv + tf_v[pl.ds(s, 16)]
            return 0
        lax.fori_loop(0, NSTR, gen_body, 0)

        for kk in range(C * 2):
            s = kk * 16
            bi = lbi_v[pl.ds(s, 16)] + xoff
            xv = plsc.load_gather(x_v, [bi])
            li_v[pl.ds(s, 16)] = xv + loff_v[pl.ds(s, 16)]

        copies = [
            pltpu.async_copy(e_hbm.at[idx_v.at[j]],
                             rows_v.at[pl.ds(j * 128, 128)], sem)
            for j in range(NSTR)
        ]
        lcp = pltpu.async_copy(w_hbm.at[li_v], wv_v, lsem)
        for cp in copies:
            cp.wait()
        lcp.wait()

        def pair_body(p, accs):
            out = []
            for b in range(C):
                e = b * EPS + 2 * p
                out.append(accs[b] + rows_v[e, :] * rows_v[e + 1, :])
            return tuple(out)
        accs = lax.fori_loop(0, NPAIR, pair_body, (zero,) * C)

        for b in range(C):
            w0 = wv_v[pl.ds(b * 32, 16)]
            w1 = wv_v[pl.ds(b * 32 + 16, 16)]
            z = jnp.sum(accs[b] + w0 + w1 * lmask + bvec)
            gv = jnp.where(lane == (c % 4) * C + b, z, gv)

        @pl.when((c % 4) == 3)
        def _store():
            o_v[pl.ds((c // 4) * 16, 16)] = gv

        return jnp.where((c % 4) == 3, zero, gv)

    lax.fori_loop(0, NCH, chunk_body, zero)

    def sig_body(k, _):
        v = o_v[pl.ds(k * 16, 16)]
        o_v[pl.ds(k * 16, 16)] = 1.0 / (1.0 + jnp.exp(-v))
        return 0
    lax.fori_loop(0, BPW // 16, sig_body, 0)

    pltpu.sync_copy(o_v, out_hbm.at[pl.ds(base, BPW)])


def kernel(x, W_lin, bias, E):
    x_flat = x.reshape(-1).astype(jnp.int32)
    w_flat = W_lin.reshape(-1)
    e_flat = E.reshape(F * V, D)
    return _build_sc_kernel()(x_flat, w_flat, bias, e_flat,
                              jnp.asarray(_TB), jnp.asarray(_TF),
                              jnp.asarray(_LBI), jnp.asarray(_LOFF))
